# Initial kernel scaffold; baseline (speedup 1.0000x reference)
import jax
import jax.numpy as jnp
from jax.experimental import pallas as pl


def kernel(z, scorer_w1, scorer_b1, scorer_w2, scorer_b2, cls_w1, cls_b1, cls_w2, cls_b2):
    raise NotImplementedError("write your pallas kernel here")


# speedup vs baseline: 1.7918x; 1.7918x over previous
import functools

import jax
import jax.numpy as jnp
from jax import lax
from jax.experimental import pallas as pl

B, T, D = 64, 8192, 128
H2 = D // 2
NUM_CLASSES = 2
K = max(1, int(T * 0.2))

BB = 8
BT = 512


def _mm(a, b):
    return jnp.dot(a.astype(jnp.bfloat16), b.astype(jnp.bfloat16),
                   preferred_element_type=jnp.float32)


def _score_seg_body(z_ref, w1_ref, b1_ref, w2_ref, sb2_ref, cw1_ref, cb1_ref,
                    cw2_ref, cb2_ref, scores_ref, seg_ref):
    z = z_ref[...].reshape(BB * BT, D)
    h = jnp.maximum(_mm(z, w1_ref[...]) + b1_ref[...], 0.0)
    s = _mm(h, w2_ref[...])[:, 0] + sb2_ref[0, 0]
    scores_ref[...] = s.reshape(BB, BT)

    h2 = jnp.maximum(_mm(z, cw1_ref[...]) + cb1_ref[...], 0.0)
    seg = _mm(h2, cw2_ref[...]) + cb2_ref[...]
    seg_ref[...] = seg.reshape(BB, BT, NUM_CLASSES)


def _select_body(scores_ref, w_ref):
    x = scores_ref[...]
    xi = lax.bitcast_convert_type(x, jnp.int32)
    key = lax.bitcast_convert_type(
        xi ^ ((xi >> 31) | jnp.int32(-2147483648)), jnp.uint32)
    t = jnp.zeros((B, 1), dtype=jnp.uint32)
    for b in range(31, -1, -1):
        cand = t | jnp.uint32(1 << b)
        cnt = jnp.sum((key >= cand).astype(jnp.int32), axis=1, keepdims=True)
        t = jnp.where(cnt >= K, cand, t)
    gt = key > t
    eq = key == t
    need = K - jnp.sum(gt.astype(jnp.int32), axis=1, keepdims=True)
    iota = lax.broadcasted_iota(jnp.int32, (B, T), 1)
    jthr = jnp.zeros((B, 1), dtype=jnp.int32)
    for b in range(13, -1, -1):
        cand = jthr | jnp.int32(1 << b)
        cnt = jnp.sum((eq & (iota < cand)).astype(jnp.int32),
                      axis=1, keepdims=True)
        ok = (cand <= T) & (cnt <= need)
        jthr = jnp.where(ok, cand, jthr)
    sel = gt | (eq & (iota < jthr))
    w_ref[...] = sel.astype(jnp.float32) * jnp.float32(1.0 / K)


def _clip_body(z_ref, w_ref, acc_ref):
    j = pl.program_id(1)

    @pl.when(j == 0)
    def _():
        acc_ref[...] = jnp.zeros_like(acc_ref)

    part = jnp.sum(z_ref[...] * w_ref[...][..., None], axis=1)
    acc_ref[...] += part


def _cls_body(x_ref, cw1_ref, cb1_ref, cw2_ref, cb2_ref, out_ref):
    h2 = jnp.maximum(_mm(x_ref[...], cw1_ref[...]) + cb1_ref[...], 0.0)
    out_ref[...] = _mm(h2, cw2_ref[...]) + cb2_ref[...]


@jax.jit
def kernel(z, scorer_w1, scorer_b1, scorer_w2, scorer_b2,
           cls_w1, cls_b1, cls_w2, cls_b2):
    b1 = scorer_b1.reshape(1, D)
    cb1 = cls_b1.reshape(1, H2)
    cb2 = cls_b2.reshape(1, NUM_CLASSES)

    grid = (B // BB, T // BT)
    scores, seg_logits = pl.pallas_call(
        _score_seg_body,
        grid=grid,
        in_specs=[
            pl.BlockSpec((BB, BT, D), lambda i, j: (i, j, 0)),
            pl.BlockSpec((D, D), lambda i, j: (0, 0)),
            pl.BlockSpec((1, D), lambda i, j: (0, 0)),
            pl.BlockSpec((D, 1), lambda i, j: (0, 0)),
            pl.BlockSpec((1, 1), lambda i, j: (0, 0)),
            pl.BlockSpec((D, H2), lambda i, j: (0, 0)),
            pl.BlockSpec((1, H2), lambda i, j: (0, 0)),
            pl.BlockSpec((H2, NUM_CLASSES), lambda i, j: (0, 0)),
            pl.BlockSpec((1, NUM_CLASSES), lambda i, j: (0, 0)),
        ],
        out_specs=[
            pl.BlockSpec((BB, BT), lambda i, j: (i, j)),
            pl.BlockSpec((BB, BT, NUM_CLASSES), lambda i, j: (i, j, 0)),
        ],
        out_shape=[
            jax.ShapeDtypeStruct((B, T), jnp.float32),
            jax.ShapeDtypeStruct((B, T, NUM_CLASSES), jnp.float32),
        ],
    )(z, scorer_w1, b1, scorer_w2, scorer_b2.reshape(1, 1),
      cls_w1, cb1, cls_w2, cb2)

    weights = pl.pallas_call(
        _select_body,
        out_shape=jax.ShapeDtypeStruct((B, T), jnp.float32),
    )(scores)

    clip_feat = pl.pallas_call(
        _clip_body,
        grid=grid,
        in_specs=[
            pl.BlockSpec((BB, BT, D), lambda i, j: (i, j, 0)),
            pl.BlockSpec((BB, BT), lambda i, j: (i, j)),
        ],
        out_specs=pl.BlockSpec((BB, D), lambda i, j: (i, 0)),
        out_shape=jax.ShapeDtypeStruct((B, D), jnp.float32),
    )(z, weights)

    clip_logits = pl.pallas_call(
        _cls_body,
        out_shape=jax.ShapeDtypeStruct((B, NUM_CLASSES), jnp.float32),
    )(clip_feat, cls_w1, cb1, cls_w2, cb2)

    return (clip_logits, seg_logits, weights, scores)


# speedup vs baseline: 2.5804x; 1.4401x over previous
import functools

import jax
import jax.numpy as jnp
from jax import lax
from jax.experimental import pallas as pl

B, T, D = 64, 8192, 128
H2 = D // 2
NUM_CLASSES = 2
K = max(1, int(T * 0.2))

BB = 8
BT = 512


def _mm(a, b):
    return jnp.dot(a.astype(jnp.bfloat16), b.astype(jnp.bfloat16),
                   preferred_element_type=jnp.float32)


def _bf(x):
    return x.astype(jnp.bfloat16).astype(jnp.float32)


def _score_seg_body(z_ref, wcat_ref, bcat_ref, m8_ref, sb2_ref,
                    cb2_ref, scores_ref, seg_ref):
    z = z_ref[...].reshape(BB * BT, D)
    g = jnp.maximum(_mm(z, wcat_ref[...]) + bcat_ref[...], 0.0)
    out = _mm(g, m8_ref[...])
    outt = out.T
    scores_ref[...] = outt[0].reshape(BB, BT) + sb2_ref[0, 0]
    seg_ref[0] = outt[1].reshape(BB, BT) + cb2_ref[0, 0]
    seg_ref[1] = outt[2].reshape(BB, BT) + cb2_ref[0, 1]


def _select_body(scores_ref, w_ref):
    x = scores_ref[...]
    xi = lax.bitcast_convert_type(x, jnp.int32)
    key = lax.bitcast_convert_type(
        xi ^ ((xi >> 31) | jnp.int32(-2147483648)), jnp.uint32)
    t = jnp.zeros((B, 1), dtype=jnp.uint32)
    for b in range(31, -1, -1):
        cand = t | jnp.uint32(1 << b)
        cnt = jnp.sum((key >= cand).astype(jnp.int32), axis=1, keepdims=True)
        t = jnp.where(cnt >= K, cand, t)
    gt = key > t
    eq = key == t
    need = K - jnp.sum(gt.astype(jnp.int32), axis=1, keepdims=True)
    iota = lax.broadcasted_iota(jnp.int32, (B, T), 1)
    jthr = jnp.zeros((B, 1), dtype=jnp.int32)
    for b in range(13, -1, -1):
        cand = jthr | jnp.int32(1 << b)
        cnt = jnp.sum((eq & (iota < cand)).astype(jnp.int32),
                      axis=1, keepdims=True)
        ok = (cand <= T) & (cnt <= need)
        jthr = jnp.where(ok, cand, jthr)
    sel = gt | (eq & (iota < jthr))
    w_ref[...] = sel.astype(jnp.float32) * jnp.float32(1.0 / K)


def _clip_body(z_ref, w_ref, acc_ref):
    j = pl.program_id(1)

    @pl.when(j == 0)
    def _():
        acc_ref[...] = jnp.zeros_like(acc_ref)

    part = jnp.sum(z_ref[...] * w_ref[...][..., None], axis=1)
    acc_ref[...] += part


def _cls_body(x_ref, cw1_ref, cb1_ref, cw2_ref, cb2_ref, out_ref):
    h2 = jnp.maximum(_mm(x_ref[...], cw1_ref[...]) + cb1_ref[...], 0.0)
    out_ref[...] = _mm(h2, cw2_ref[...]) + cb2_ref[...]


@jax.jit
def kernel(z, scorer_w1, scorer_b1, scorer_w2, scorer_b2,
           cls_w1, cls_b1, cls_w2, cls_b2):
    b1 = scorer_b1.reshape(1, D)
    cb1 = cls_b1.reshape(1, H2)
    cb2 = cls_b2.reshape(1, NUM_CLASSES)
    wcat = jnp.concatenate([scorer_w1, cls_w1], axis=1)
    bcat = jnp.concatenate([b1, cb1], axis=1)
    m8 = jnp.zeros((D + H2, 8), jnp.float32)
    m8 = m8.at[:D, 0].set(scorer_w2[:, 0])
    m8 = m8.at[D:, 1].set(cls_w2[:, 0])
    m8 = m8.at[D:, 2].set(cls_w2[:, 1])

    grid = (B // BB, T // BT)
    scores, seg_t = pl.pallas_call(
        _score_seg_body,
        grid=grid,
        in_specs=[
            pl.BlockSpec((BB, BT, D), lambda i, j: (i, j, 0)),
            pl.BlockSpec((D, D + H2), lambda i, j: (0, 0)),
            pl.BlockSpec((1, D + H2), lambda i, j: (0, 0)),
            pl.BlockSpec((D + H2, 8), lambda i, j: (0, 0)),
            pl.BlockSpec((1, 1), lambda i, j: (0, 0)),
            pl.BlockSpec((1, NUM_CLASSES), lambda i, j: (0, 0)),
        ],
        out_specs=[
            pl.BlockSpec((BB, BT), lambda i, j: (i, j)),
            pl.BlockSpec((NUM_CLASSES, BB, BT), lambda i, j: (0, i, j)),
        ],
        out_shape=[
            jax.ShapeDtypeStruct((B, T), jnp.float32),
            jax.ShapeDtypeStruct((NUM_CLASSES, B, T), jnp.float32),
        ],
    )(z, wcat, bcat, m8, scorer_b2.reshape(1, 1), cb2)
    seg_logits = jnp.moveaxis(seg_t, 0, 2)

    weights = pl.pallas_call(
        _select_body,
        out_shape=jax.ShapeDtypeStruct((B, T), jnp.float32),
    )(scores)

    clip_feat = pl.pallas_call(
        _clip_body,
        grid=grid,
        in_specs=[
            pl.BlockSpec((BB, BT, D), lambda i, j: (i, j, 0)),
            pl.BlockSpec((BB, BT), lambda i, j: (i, j)),
        ],
        out_specs=pl.BlockSpec((BB, D), lambda i, j: (i, 0)),
        out_shape=jax.ShapeDtypeStruct((B, D), jnp.float32),
    )(z, weights)

    clip_logits = pl.pallas_call(
        _cls_body,
        out_shape=jax.ShapeDtypeStruct((B, NUM_CLASSES), jnp.float32),
    )(clip_feat, cls_w1, cb1, cls_w2, cb2)

    return (clip_logits, seg_logits, weights, scores)


# speedup vs baseline: 3.3163x; 1.2852x over previous
import functools

import jax
import jax.numpy as jnp
from jax import lax
from jax.experimental import pallas as pl

B, T, D = 64, 8192, 128
H2 = D // 2
NUM_CLASSES = 2
K = max(1, int(T * 0.2))

BB = 8
BT = 1024
BTC = 2048


def _mm(a, b):
    return jnp.dot(a.astype(jnp.bfloat16), b.astype(jnp.bfloat16),
                   preferred_element_type=jnp.float32)


def _bf(x):
    return x.astype(jnp.bfloat16).astype(jnp.float32)


def _score_seg_body(z_ref, wcat_ref, bcat_ref, m8_ref, sb2_ref,
                    cb2_ref, scores_ref, seg_ref):
    z = z_ref[...].reshape(BB * BT, D)
    g = jnp.maximum(_mm(z, wcat_ref[...]) + bcat_ref[...], 0.0)
    out = _mm(g, m8_ref[...])
    outt = out.T
    scores_ref[...] = outt[0].reshape(BB, BT) + sb2_ref[0, 0]
    seg_ref[0] = outt[1].reshape(BB, BT) + cb2_ref[0, 0]
    seg_ref[1] = outt[2].reshape(BB, BT) + cb2_ref[0, 1]


def _select_body(scores_ref, w_ref):
    x = scores_ref[...]
    xi = lax.bitcast_convert_type(x, jnp.int32)
    key = lax.bitcast_convert_type(
        xi ^ ((xi >> 31) | jnp.int32(-2147483648)), jnp.uint32)
    t = jnp.zeros((B, 1), dtype=jnp.uint32)
    for b in range(31, -1, -1):
        cand = t | jnp.uint32(1 << b)
        cnt = jnp.sum((key >= cand).astype(jnp.int32), axis=1, keepdims=True)
        t = jnp.where(cnt >= K, cand, t)
    gt = key > t
    eq = key == t
    need = K - jnp.sum(gt.astype(jnp.int32), axis=1, keepdims=True)
    iota = lax.broadcasted_iota(jnp.int32, (B, T), 1)
    jthr = jnp.zeros((B, 1), dtype=jnp.int32)
    for b in range(13, -1, -1):
        cand = jthr | jnp.int32(1 << b)
        cnt = jnp.sum((eq & (iota < cand)).astype(jnp.int32),
                      axis=1, keepdims=True)
        ok = (cand <= T) & (cnt <= need)
        jthr = jnp.where(ok, cand, jthr)
    sel = gt | (eq & (iota < jthr))
    w_ref[...] = sel.astype(jnp.float32) * jnp.float32(1.0 / K)


def _clip_body(z_ref, w_ref, acc_ref):
    j = pl.program_id(1)

    @pl.when(j == 0)
    def _():
        acc_ref[...] = jnp.zeros_like(acc_ref)

    part = jnp.sum(z_ref[...] * w_ref[...][..., None], axis=1)
    acc_ref[...] += part


def _cls_body(x_ref, cw1_ref, cb1_ref, cw2_ref, cb2_ref, out_ref):
    h2 = jnp.maximum(_mm(x_ref[...], cw1_ref[...]) + cb1_ref[...], 0.0)
    out_ref[...] = _mm(h2, cw2_ref[...]) + cb2_ref[...]


@jax.jit
def kernel(z, scorer_w1, scorer_b1, scorer_w2, scorer_b2,
           cls_w1, cls_b1, cls_w2, cls_b2):
    b1 = scorer_b1.reshape(1, D)
    cb1 = cls_b1.reshape(1, H2)
    cb2 = cls_b2.reshape(1, NUM_CLASSES)
    wcat = jnp.concatenate([scorer_w1, cls_w1], axis=1)
    bcat = jnp.concatenate([b1, cb1], axis=1)
    m8 = jnp.zeros((D + H2, 8), jnp.float32)
    m8 = m8.at[:D, 0].set(scorer_w2[:, 0])
    m8 = m8.at[D:, 1].set(cls_w2[:, 0])
    m8 = m8.at[D:, 2].set(cls_w2[:, 1])

    grid = (B // BB, T // BT)
    scores, seg_t = pl.pallas_call(
        _score_seg_body,
        grid=grid,
        in_specs=[
            pl.BlockSpec((BB, BT, D), lambda i, j: (i, j, 0)),
            pl.BlockSpec((D, D + H2), lambda i, j: (0, 0)),
            pl.BlockSpec((1, D + H2), lambda i, j: (0, 0)),
            pl.BlockSpec((D + H2, 8), lambda i, j: (0, 0)),
            pl.BlockSpec((1, 1), lambda i, j: (0, 0)),
            pl.BlockSpec((1, NUM_CLASSES), lambda i, j: (0, 0)),
        ],
        out_specs=[
            pl.BlockSpec((BB, BT), lambda i, j: (i, j)),
            pl.BlockSpec((NUM_CLASSES, BB, BT), lambda i, j: (0, i, j)),
        ],
        out_shape=[
            jax.ShapeDtypeStruct((B, T), jnp.float32),
            jax.ShapeDtypeStruct((NUM_CLASSES, B, T), jnp.float32),
        ],
    )(z, wcat, bcat, m8, scorer_b2.reshape(1, 1), cb2)
    seg_logits = jnp.moveaxis(seg_t, 0, 2)

    weights = pl.pallas_call(
        _select_body,
        out_shape=jax.ShapeDtypeStruct((B, T), jnp.float32),
    )(scores)

    clip_feat = pl.pallas_call(
        _clip_body,
        grid=(B // BB, T // BTC),
        in_specs=[
            pl.BlockSpec((BB, BTC, D), lambda i, j: (i, j, 0)),
            pl.BlockSpec((BB, BTC), lambda i, j: (i, j)),
        ],
        out_specs=pl.BlockSpec((BB, D), lambda i, j: (i, 0)),
        out_shape=jax.ShapeDtypeStruct((B, D), jnp.float32),
    )(z, weights)

    clip_logits = pl.pallas_call(
        _cls_body,
        out_shape=jax.ShapeDtypeStruct((B, NUM_CLASSES), jnp.float32),
    )(clip_feat, cls_w1, cb1, cls_w2, cb2)

    return (clip_logits, seg_logits, weights, scores)


# speedup vs baseline: 3.5225x; 1.0622x over previous
import functools

import jax
import jax.numpy as jnp
from jax import lax
from jax.experimental import pallas as pl

B, T, D = 64, 8192, 128
H2 = D // 2
NUM_CLASSES = 2
K = max(1, int(T * 0.2))

BB = 8
BT = 2048
BTC = 4096


def _mm(a, b):
    return jnp.dot(a.astype(jnp.bfloat16), b.astype(jnp.bfloat16),
                   preferred_element_type=jnp.float32)


def _bf(x):
    return x.astype(jnp.bfloat16).astype(jnp.float32)


def _score_seg_body(z_ref, wcat_ref, bcat_ref, m8_ref, sb2_ref,
                    cb2_ref, scores_ref, seg_ref):
    z = z_ref[...].reshape(BB * BT, D)
    g = jnp.maximum(_mm(z, wcat_ref[...]) + bcat_ref[...], 0.0)
    out = _mm(g, m8_ref[...])
    outt = out.T
    scores_ref[...] = outt[0].reshape(BB, BT) + sb2_ref[0, 0]
    seg_ref[0] = outt[1].reshape(BB, BT) + cb2_ref[0, 0]
    seg_ref[1] = outt[2].reshape(BB, BT) + cb2_ref[0, 1]


def _select_body(scores_ref, w_ref):
    x = scores_ref[...]
    xi = lax.bitcast_convert_type(x, jnp.int32)
    key = lax.bitcast_convert_type(
        xi ^ ((xi >> 31) | jnp.int32(-2147483648)), jnp.uint32)
    t = jnp.zeros((B, 1), dtype=jnp.uint32)
    for b in range(31, -1, -1):
        cand = t | jnp.uint32(1 << b)
        cnt = jnp.sum((key >= cand).astype(jnp.int32), axis=1, keepdims=True)
        t = jnp.where(cnt >= K, cand, t)
    gt = key > t
    eq = key == t
    need = K - jnp.sum(gt.astype(jnp.int32), axis=1, keepdims=True)
    iota = lax.broadcasted_iota(jnp.int32, (B, T), 1)
    jthr = jnp.zeros((B, 1), dtype=jnp.int32)
    for b in range(13, -1, -1):
        cand = jthr | jnp.int32(1 << b)
        cnt = jnp.sum((eq & (iota < cand)).astype(jnp.int32),
                      axis=1, keepdims=True)
        ok = (cand <= T) & (cnt <= need)
        jthr = jnp.where(ok, cand, jthr)
    sel = gt | (eq & (iota < jthr))
    w_ref[...] = sel.astype(jnp.float32) * jnp.float32(1.0 / K)


def _clip_body(z_ref, w_ref, acc_ref):
    j = pl.program_id(1)

    @pl.when(j == 0)
    def _():
        acc_ref[...] = jnp.zeros_like(acc_ref)

    part = jnp.sum(z_ref[...] * w_ref[...][..., None], axis=1)
    acc_ref[...] += part


def _cls_body(x_ref, cw1_ref, cb1_ref, cw2_ref, cb2_ref, out_ref):
    h2 = jnp.maximum(_mm(x_ref[...], cw1_ref[...]) + cb1_ref[...], 0.0)
    out_ref[...] = _mm(h2, cw2_ref[...]) + cb2_ref[...]


@jax.jit
def kernel(z, scorer_w1, scorer_b1, scorer_w2, scorer_b2,
           cls_w1, cls_b1, cls_w2, cls_b2):
    b1 = scorer_b1.reshape(1, D)
    cb1 = cls_b1.reshape(1, H2)
    cb2 = cls_b2.reshape(1, NUM_CLASSES)
    wcat = jnp.concatenate([scorer_w1, cls_w1], axis=1)
    bcat = jnp.concatenate([b1, cb1], axis=1)
    m8 = jnp.zeros((D + H2, 8), jnp.float32)
    m8 = m8.at[:D, 0].set(scorer_w2[:, 0])
    m8 = m8.at[D:, 1].set(cls_w2[:, 0])
    m8 = m8.at[D:, 2].set(cls_w2[:, 1])

    grid = (B // BB, T // BT)
    scores, seg_t = pl.pallas_call(
        _score_seg_body,
        grid=grid,
        in_specs=[
            pl.BlockSpec((BB, BT, D), lambda i, j: (i, j, 0)),
            pl.BlockSpec((D, D + H2), lambda i, j: (0, 0)),
            pl.BlockSpec((1, D + H2), lambda i, j: (0, 0)),
            pl.BlockSpec((D + H2, 8), lambda i, j: (0, 0)),
            pl.BlockSpec((1, 1), lambda i, j: (0, 0)),
            pl.BlockSpec((1, NUM_CLASSES), lambda i, j: (0, 0)),
        ],
        out_specs=[
            pl.BlockSpec((BB, BT), lambda i, j: (i, j)),
            pl.BlockSpec((NUM_CLASSES, BB, BT), lambda i, j: (0, i, j)),
        ],
        out_shape=[
            jax.ShapeDtypeStruct((B, T), jnp.float32),
            jax.ShapeDtypeStruct((NUM_CLASSES, B, T), jnp.float32),
        ],
    )(z, wcat, bcat, m8, scorer_b2.reshape(1, 1), cb2)
    seg_logits = jnp.moveaxis(seg_t, 0, 2)

    weights = pl.pallas_call(
        _select_body,
        out_shape=jax.ShapeDtypeStruct((B, T), jnp.float32),
    )(scores)

    clip_feat = pl.pallas_call(
        _clip_body,
        grid=(B // BB, T // BTC),
        in_specs=[
            pl.BlockSpec((BB, BTC, D), lambda i, j: (i, j, 0)),
            pl.BlockSpec((BB, BTC), lambda i, j: (i, j)),
        ],
        out_specs=pl.BlockSpec((BB, D), lambda i, j: (i, 0)),
        out_shape=jax.ShapeDtypeStruct((B, D), jnp.float32),
    )(z, weights)

    clip_logits = pl.pallas_call(
        _cls_body,
        out_shape=jax.ShapeDtypeStruct((B, NUM_CLASSES), jnp.float32),
    )(clip_feat, cls_w1, cb1, cls_w2, cb2)

    return (clip_logits, seg_logits, weights, scores)


# speedup vs baseline: 3.5638x; 1.0117x over previous
import functools

import jax
import jax.numpy as jnp
from jax import lax
from jax.experimental import pallas as pl

B, T, D = 64, 8192, 128
H2 = D // 2
NUM_CLASSES = 2
K = max(1, int(T * 0.2))

BB = 8
BT = 4096
BTC = 4096


def _mm(a, b):
    return jnp.dot(a.astype(jnp.bfloat16), b.astype(jnp.bfloat16),
                   preferred_element_type=jnp.float32)


def _bf(x):
    return x.astype(jnp.bfloat16).astype(jnp.float32)


def _score_seg_body(z_ref, wcat_ref, bcat_ref, m8_ref, sb2_ref,
                    cb2_ref, scores_ref, seg_ref):
    z = z_ref[...].reshape(BB * BT, D)
    g = jnp.maximum(_mm(z, wcat_ref[...]) + bcat_ref[...], 0.0)
    out = _mm(g, m8_ref[...])
    outt = out.T
    scores_ref[...] = outt[0].reshape(BB, BT) + sb2_ref[0, 0]
    seg_ref[0] = outt[1].reshape(BB, BT) + cb2_ref[0, 0]
    seg_ref[1] = outt[2].reshape(BB, BT) + cb2_ref[0, 1]


def _select_body(scores_ref, w_ref):
    x = scores_ref[...]
    xi = lax.bitcast_convert_type(x, jnp.int32)
    key = lax.bitcast_convert_type(
        xi ^ ((xi >> 31) | jnp.int32(-2147483648)), jnp.uint32)
    t = jnp.zeros((B, 1), dtype=jnp.uint32)
    for b in range(31, -1, -1):
        cand = t | jnp.uint32(1 << b)
        cnt = jnp.sum((key >= cand).astype(jnp.int32), axis=1, keepdims=True)
        t = jnp.where(cnt >= K, cand, t)
    gt = key > t
    eq = key == t
    need = K - jnp.sum(gt.astype(jnp.int32), axis=1, keepdims=True)
    iota = lax.broadcasted_iota(jnp.int32, (B, T), 1)
    jthr = jnp.zeros((B, 1), dtype=jnp.int32)
    for b in range(13, -1, -1):
        cand = jthr | jnp.int32(1 << b)
        cnt = jnp.sum((eq & (iota < cand)).astype(jnp.int32),
                      axis=1, keepdims=True)
        ok = (cand <= T) & (cnt <= need)
        jthr = jnp.where(ok, cand, jthr)
    sel = gt | (eq & (iota < jthr))
    w_ref[...] = sel.astype(jnp.float32) * jnp.float32(1.0 / K)


def _clip_body(z_ref, w_ref, acc_ref):
    j = pl.program_id(1)

    @pl.when(j == 0)
    def _():
        acc_ref[...] = jnp.zeros_like(acc_ref)

    part = jnp.sum(z_ref[...] * w_ref[...][..., None], axis=1)
    acc_ref[...] += part


def _cls_body(x_ref, cw1_ref, cb1_ref, cw2_ref, cb2_ref, out_ref):
    h2 = jnp.maximum(_mm(x_ref[...], cw1_ref[...]) + cb1_ref[...], 0.0)
    out_ref[...] = _mm(h2, cw2_ref[...]) + cb2_ref[...]


@jax.jit
def kernel(z, scorer_w1, scorer_b1, scorer_w2, scorer_b2,
           cls_w1, cls_b1, cls_w2, cls_b2):
    b1 = scorer_b1.reshape(1, D)
    cb1 = cls_b1.reshape(1, H2)
    cb2 = cls_b2.reshape(1, NUM_CLASSES)
    wcat = jnp.concatenate([scorer_w1, cls_w1], axis=1)
    bcat = jnp.concatenate([b1, cb1], axis=1)
    m8 = jnp.zeros((D + H2, 8), jnp.float32)
    m8 = m8.at[:D, 0].set(scorer_w2[:, 0])
    m8 = m8.at[D:, 1].set(cls_w2[:, 0])
    m8 = m8.at[D:, 2].set(cls_w2[:, 1])

    grid = (B // BB, T // BT)
    scores, seg_t = pl.pallas_call(
        _score_seg_body,
        grid=grid,
        in_specs=[
            pl.BlockSpec((BB, BT, D), lambda i, j: (i, j, 0)),
            pl.BlockSpec((D, D + H2), lambda i, j: (0, 0)),
            pl.BlockSpec((1, D + H2), lambda i, j: (0, 0)),
            pl.BlockSpec((D + H2, 8), lambda i, j: (0, 0)),
            pl.BlockSpec((1, 1), lambda i, j: (0, 0)),
            pl.BlockSpec((1, NUM_CLASSES), lambda i, j: (0, 0)),
        ],
        out_specs=[
            pl.BlockSpec((BB, BT), lambda i, j: (i, j)),
            pl.BlockSpec((NUM_CLASSES, BB, BT), lambda i, j: (0, i, j)),
        ],
        out_shape=[
            jax.ShapeDtypeStruct((B, T), jnp.float32),
            jax.ShapeDtypeStruct((NUM_CLASSES, B, T), jnp.float32),
        ],
    )(z, wcat, bcat, m8, scorer_b2.reshape(1, 1), cb2)
    seg_logits = jnp.moveaxis(seg_t, 0, 2)

    weights = pl.pallas_call(
        _select_body,
        out_shape=jax.ShapeDtypeStruct((B, T), jnp.float32),
    )(scores)

    clip_feat = pl.pallas_call(
        _clip_body,
        grid=(B // BB, T // BTC),
        in_specs=[
            pl.BlockSpec((BB, BTC, D), lambda i, j: (i, j, 0)),
            pl.BlockSpec((BB, BTC), lambda i, j: (i, j)),
        ],
        out_specs=pl.BlockSpec((BB, D), lambda i, j: (i, 0)),
        out_shape=jax.ShapeDtypeStruct((B, D), jnp.float32),
    )(z, weights)

    clip_logits = pl.pallas_call(
        _cls_body,
        out_shape=jax.ShapeDtypeStruct((B, NUM_CLASSES), jnp.float32),
    )(clip_feat, cls_w1, cb1, cls_w2, cb2)

    return (clip_logits, seg_logits, weights, scores)


# speedup vs baseline: 3.6041x; 1.0113x over previous
import functools

import jax
import jax.numpy as jnp
from jax import lax
from jax.experimental import pallas as pl
from jax.experimental.pallas import tpu as pltpu

B, T, D = 64, 8192, 128
H2 = D // 2
NUM_CLASSES = 2
K = max(1, int(T * 0.2))

BB = 8
BT = 4096
BTC = 4096


def _mm(a, b):
    return jnp.dot(a.astype(jnp.bfloat16), b.astype(jnp.bfloat16),
                   preferred_element_type=jnp.float32)


def _bf(x):
    return x.astype(jnp.bfloat16).astype(jnp.float32)


def _score_seg_body(z_ref, wcat_ref, bcat_ref, m8_ref, sb2_ref,
                    cb2_ref, scores_ref, seg_ref):
    z = z_ref[...].reshape(BB * BT, D)
    g = jnp.maximum(_mm(z, wcat_ref[...]) + bcat_ref[...], 0.0)
    out = _mm(g, m8_ref[...])
    outt = out.T
    scores_ref[...] = outt[0].reshape(BB, BT) + sb2_ref[0, 0]
    seg_ref[0] = outt[1].reshape(BB, BT) + cb2_ref[0, 0]
    seg_ref[1] = outt[2].reshape(BB, BT) + cb2_ref[0, 1]


def _select_weights(x):
    xi = lax.bitcast_convert_type(x, jnp.int32)
    key = lax.bitcast_convert_type(
        xi ^ ((xi >> 31) | jnp.int32(-2147483648)), jnp.uint32)
    t = jnp.zeros((B, 1), dtype=jnp.uint32)
    for b in range(31, -1, -1):
        cand = t | jnp.uint32(1 << b)
        cnt = jnp.sum((key >= cand).astype(jnp.int32), axis=1, keepdims=True)
        t = jnp.where(cnt >= K, cand, t)
    gt = key > t
    eq = key == t
    need = K - jnp.sum(gt.astype(jnp.int32), axis=1, keepdims=True)
    iota = lax.broadcasted_iota(jnp.int32, (B, T), 1)
    jthr = jnp.zeros((B, 1), dtype=jnp.int32)
    for b in range(13, -1, -1):
        cand = jthr | jnp.int32(1 << b)
        cnt = jnp.sum((eq & (iota < cand)).astype(jnp.int32),
                      axis=1, keepdims=True)
        ok = (cand <= T) & (cnt <= need)
        jthr = jnp.where(ok, cand, jthr)
    sel = gt | (eq & (iota < jthr))
    return sel.astype(jnp.float32) * jnp.float32(1.0 / K)


def _clip_body(scores_ref, z_ref, w_ref, acc_ref, w_scr):
    i = pl.program_id(0)
    j = pl.program_id(1)

    @pl.when((i == 0) & (j == 0))
    def _():
        w = _select_weights(scores_ref[...])
        w_scr[...] = w
        w_ref[...] = w

    @pl.when(j == 0)
    def _():
        acc_ref[...] = jnp.zeros_like(acc_ref)

    wblk = w_scr[pl.ds(i * BB, BB), pl.ds(j * BTC, BTC)]
    part = jnp.sum(z_ref[...] * wblk[..., None], axis=1)
    acc_ref[...] += part


def _cls_body(x_ref, cw1_ref, cb1_ref, cw2_ref, cb2_ref, out_ref):
    h2 = jnp.maximum(_mm(x_ref[...], cw1_ref[...]) + cb1_ref[...], 0.0)
    out_ref[...] = _mm(h2, cw2_ref[...]) + cb2_ref[...]


@jax.jit
def kernel(z, scorer_w1, scorer_b1, scorer_w2, scorer_b2,
           cls_w1, cls_b1, cls_w2, cls_b2):
    b1 = scorer_b1.reshape(1, D)
    cb1 = cls_b1.reshape(1, H2)
    cb2 = cls_b2.reshape(1, NUM_CLASSES)
    wcat = jnp.concatenate([scorer_w1, cls_w1], axis=1)
    bcat = jnp.concatenate([b1, cb1], axis=1)
    m8 = jnp.zeros((D + H2, 8), jnp.float32)
    m8 = m8.at[:D, 0].set(scorer_w2[:, 0])
    m8 = m8.at[D:, 1].set(cls_w2[:, 0])
    m8 = m8.at[D:, 2].set(cls_w2[:, 1])

    grid = (B // BB, T // BT)
    scores, seg_t = pl.pallas_call(
        _score_seg_body,
        grid=grid,
        in_specs=[
            pl.BlockSpec((BB, BT, D), lambda i, j: (i, j, 0)),
            pl.BlockSpec((D, D + H2), lambda i, j: (0, 0)),
            pl.BlockSpec((1, D + H2), lambda i, j: (0, 0)),
            pl.BlockSpec((D + H2, 8), lambda i, j: (0, 0)),
            pl.BlockSpec((1, 1), lambda i, j: (0, 0)),
            pl.BlockSpec((1, NUM_CLASSES), lambda i, j: (0, 0)),
        ],
        out_specs=[
            pl.BlockSpec((BB, BT), lambda i, j: (i, j)),
            pl.BlockSpec((NUM_CLASSES, BB, BT), lambda i, j: (0, i, j)),
        ],
        out_shape=[
            jax.ShapeDtypeStruct((B, T), jnp.float32),
            jax.ShapeDtypeStruct((NUM_CLASSES, B, T), jnp.float32),
        ],
    )(z, wcat, bcat, m8, scorer_b2.reshape(1, 1), cb2)
    seg_logits = jnp.moveaxis(seg_t, 0, 2)

    weights, clip_feat = pl.pallas_call(
        _clip_body,
        grid=(B // BB, T // BTC),
        in_specs=[
            pl.BlockSpec((B, T), lambda i, j: (0, 0)),
            pl.BlockSpec((BB, BTC, D), lambda i, j: (i, j, 0)),
        ],
        out_specs=[
            pl.BlockSpec((B, T), lambda i, j: (0, 0)),
            pl.BlockSpec((BB, D), lambda i, j: (i, 0)),
        ],
        out_shape=[
            jax.ShapeDtypeStruct((B, T), jnp.float32),
            jax.ShapeDtypeStruct((B, D), jnp.float32),
        ],
        scratch_shapes=[pltpu.VMEM((B, T), jnp.float32)],
    )(scores, z)

    clip_logits = pl.pallas_call(
        _cls_body,
        out_shape=jax.ShapeDtypeStruct((B, NUM_CLASSES), jnp.float32),
    )(clip_feat, cls_w1, cb1, cls_w2, cb2)

    return (clip_logits, seg_logits, weights, scores)
